# baseline (speedup 1.0000x reference)
import functools

import jax
import jax.numpy as jnp
from jax import lax
from jax.experimental import pallas as pl
from jax.experimental.pallas import tpu as pltpu
from jax.experimental.pallas import tpu_sc as plsc

L = 16
TF = 8
TV = 128


def _iota16():
    return lax.iota(jnp.int32, L)


@functools.lru_cache(maxsize=None)
def _make_detile(V: int, D: int):
    info = plsc.get_sparse_core_info()
    NC, NS = info.num_cores, info.num_subcores
    NW = NC * NS
    n_vb = -(-V // TV)
    n_full = V // TV
    v_tail = V - n_full * TV
    n_fb = D // TF
    n_iter = -(-n_full // NW)
    NB = 2
    band_words = TV * D

    mesh = plsc.VectorSubcoreMesh(core_axis_name="c", subcore_axis_name="s")

    @functools.partial(
        pl.kernel,
        mesh=mesh,
        out_type=jax.ShapeDtypeStruct((n_vb * band_words,), jnp.float32),
        scratch_types=[pltpu.VMEM((TF, TV), jnp.float32)] * (NB * n_fb)
        + [pltpu.VMEM((band_words,), jnp.float32)] * NB
        + [pltpu.SemaphoreType.DMA] * (2 * NB),
        compiler_params=pltpu.CompilerParams(needs_layout_passes=False),
    )
    def detile_kernel(t_hbm, r_hbm, *scratch):
        tiles = [scratch[b * n_fb : (b + 1) * n_fb] for b in range(NB)]
        buft = scratch[NB * n_fb : NB * n_fb + NB]
        isem = scratch[NB * n_fb + NB : NB * n_fb + 2 * NB]
        wsem = scratch[NB * n_fb + 2 * NB :]
        wid = lax.axis_index("s") * NC + lax.axis_index("c")

        def vb_of(j):
            return jnp.minimum(wid + j * NW, n_full - 1)

        def start_fill(j, b):
            vb = vb_of(j)
            for fb in range(n_fb):
                pltpu.async_copy(
                    t_hbm.at[pl.ds(fb * TF, TF), pl.ds(vb * TV, TV)],
                    tiles[b][fb],
                    isem[b],
                )

        def wait_fill(b):
            for fb in range(n_fb):
                pltpu.make_async_copy(
                    t_hbm.at[pl.ds(0, TF), pl.ds(0, TV)], tiles[b][fb], isem[b]
                ).wait()

        def start_write(j, b):
            pltpu.async_copy(
                buft[b],
                r_hbm.at[pl.ds(vb_of(j) * band_words, band_words)],
                wsem[b],
            )

        def wait_write(b):
            pltpu.make_async_copy(
                buft[b], r_hbm.at[pl.ds(0, band_words)], wsem[b]
            ).wait()

        def transpose_band(bs, n_v):
            for fb in range(n_fb):
                for fi in range(TF):
                    f = fb * TF + fi
                    for k in range(n_v // L):
                        vals = tiles[bs][fb][fi, pl.ds(k * L, L)]
                        plsc.store_scatter(
                            buft[bs], [_iota16() * D + (k * L * D + f)], vals
                        )

        for b in range(NB):
            start_fill(b, b)

        def body(j, carry):
            bsel = lax.rem(j, NB)

            def for_buf(bs):
                @pl.when(bsel == bs)
                def _():
                    wait_fill(bs)

                    @pl.when(j >= NB)
                    def _():
                        wait_write(bs)

                    transpose_band(bs, TV)

                    @pl.when(j + NB < n_iter)
                    def _():
                        start_fill(j + NB, bs)

                    start_write(j, bs)

            for bs in range(NB):
                for_buf(bs)
            return carry

        lax.fori_loop(0, n_iter, body, 0)
        for b in range(NB):
            wait_write(b)

    return detile_kernel


@functools.lru_cache(maxsize=None)
def _make_tail_fix(n_r: int, off: int, n_tail: int):
    def tail_kernel(r_in, tail_in, r_out, tmp, s1, s2):
        cp1 = pltpu.make_async_copy(tail_in, tmp, s1)
        cp1.start()
        cp1.wait()
        cp2 = pltpu.make_async_copy(tmp, r_out.at[pl.ds(off, n_tail)], s2)
        cp2.start()
        cp2.wait()

    return pl.pallas_call(
        tail_kernel,
        out_shape=jax.ShapeDtypeStruct((n_r,), jnp.float32),
        in_specs=[
            pl.BlockSpec(memory_space=pl.ANY),
            pl.BlockSpec(memory_space=pl.ANY),
        ],
        out_specs=pl.BlockSpec(memory_space=pl.ANY),
        input_output_aliases={0: 0},
        scratch_shapes=[
            pltpu.VMEM((n_tail,), jnp.float32),
            pltpu.SemaphoreType.DMA,
            pltpu.SemaphoreType.DMA,
        ],
    )


@functools.lru_cache(maxsize=None)
def _make_gather(Vpad: int, D: int, T: int, B0: int):
    info = plsc.get_sparse_core_info()
    NC, NS = info.num_cores, info.num_subcores
    NW = NC * NS
    n_bb = B0 // TV
    n_chunks = (T * n_bb) // NW
    NB = 2
    n_fb = D // TF

    mesh = plsc.VectorSubcoreMesh(core_axis_name="c", subcore_axis_name="s")

    @functools.partial(
        pl.kernel,
        mesh=mesh,
        out_type=jax.ShapeDtypeStruct((T, n_fb, n_bb, TF, TV), jnp.float32),
        scratch_types=[
            pltpu.VMEM((NB, TV), jnp.int32),
            pltpu.VMEM((NB, TV, D), jnp.float32),
            pltpu.VMEM((NB, n_fb, TF, TV), jnp.float32),
        ]
        + [pltpu.SemaphoreType.DMA] * (3 * NB),
        compiler_params=pltpu.CompilerParams(
            use_tc_tiling_on_sc=False, needs_layout_passes=False
        ),
    )
    def gather_kernel(x4_hbm, r_hbm, out_hbm, idx_v, g, gt, *sems):
        xsem = sems[:NB]
        gsem = sems[NB : 2 * NB]
        wsem = sems[2 * NB :]
        wid = lax.axis_index("s") * NC + lax.axis_index("c")
        base = wid * n_chunks

        def tb_of(i):
            c = base + i
            return lax.div(c, n_bb), lax.rem(c, n_bb)

        def start_idx(i, b):
            t, bb = tb_of(i)
            pltpu.async_copy(
                x4_hbm.at[lax.shift_right_logical(t, 3), bb, lax.rem(t, TF)],
                idx_v.at[b],
                xsem[b],
            )

        def wait_idx(b):
            pltpu.make_async_copy(
                x4_hbm.at[0, 0, 0], idx_v.at[b], xsem[b]
            ).wait()

        def start_gather(b):
            pltpu.async_copy(r_hbm.at[idx_v.at[b]], g.at[b], gsem[b])

        def wait_gather(b):
            pltpu.make_async_copy(
                r_hbm.at[idx_v.at[b]], g.at[b], gsem[b]
            ).wait()

        def start_write(i, b):
            t, bb = tb_of(i)
            pltpu.async_copy(gt.at[b], out_hbm.at[t, :, bb], wsem[b])

        def wait_write(b):
            pltpu.make_async_copy(
                gt.at[b], out_hbm.at[0, :, 0], wsem[b]
            ).wait()

        for b in range(NB):
            start_idx(b, b)
        for b in range(NB):
            wait_idx(b)
            start_gather(b)

        def body(i, carry):
            bsel = lax.rem(i, NB)

            def for_buf(bs):
                @pl.when(bsel == bs)
                def _():
                    wait_gather(bs)

                    @pl.when(i + NB < n_chunks)
                    def _():
                        start_idx(i + NB, bs)

                    @pl.when(i >= NB)
                    def _():
                        wait_write(bs)

                    def tr(f, c):
                        fb = lax.shift_right_logical(f, 3)
                        fi = lax.rem(f, TF)
                        for k in range(TV // L):
                            vals = plsc.load_gather(
                                g.at[bs],
                                [
                                    k * L + _iota16(),
                                    jnp.full((L,), 0, jnp.int32) + f,
                                ],
                            )
                            gt.at[bs][fb, fi, pl.ds(k * L, L)] = vals
                        return c

                    lax.fori_loop(0, D, tr, 0)

                    @pl.when(i + NB < n_chunks)
                    def _():
                        wait_idx(bs)
                        start_gather(bs)

                    start_write(i, bs)

            for bs in range(NB):
                for_buf(bs)
            return carry

        lax.fori_loop(0, n_chunks, body, 0)
        for b in range(NB):
            wait_write(b)

    return gather_kernel


@jax.jit
def kernel(x, table):
    n_rows, n_cols = x.shape
    V, D = table.shape
    n_vb = -(-V // TV)
    t_phys = table.T
    x4 = (
        x.T.astype(jnp.int32)
        .reshape(n_cols // TF, TF, n_rows // TV, TV)
        .transpose(0, 2, 1, 3)
    )
    n_full = V // TV
    r_flat = _make_detile(V, D)(t_phys)
    if V % TV:
        tail = lax.slice(table, (n_full * TV, 0), (V, D)).reshape(-1)
        r_flat = _make_tail_fix(
            r_flat.shape[0], n_full * TV * D, tail.shape[0]
        )(r_flat, tail)
    r = r_flat.reshape(n_vb * TV, D)
    out5 = _make_gather(n_vb * TV, D, n_cols, n_rows)(x4, r)
    return out5.transpose(2, 4, 0, 1, 3).reshape(n_rows, n_cols, D)


# speedup vs baseline: 2.2445x; 2.2445x over previous
import functools

import jax
import jax.numpy as jnp
from jax import lax
from jax.experimental import pallas as pl
from jax.experimental.pallas import tpu as pltpu
from jax.experimental.pallas import tpu_sc as plsc

L = 16
TF = 8
TV = 128


def _iota16():
    return lax.iota(jnp.int32, L)


@functools.lru_cache(maxsize=None)
def _make_detile(V: int, D: int):
    info = plsc.get_sparse_core_info()
    NC, NS = info.num_cores, info.num_subcores
    NW = NC * NS
    n_vb = -(-V // TV)
    n_full = V // TV
    v_tail = V - n_full * TV
    n_fb = D // TF
    n_iter = -(-n_full // NW)
    NB = 2
    band_words = TV * D

    mesh = plsc.VectorSubcoreMesh(core_axis_name="c", subcore_axis_name="s")

    @functools.partial(
        pl.kernel,
        mesh=mesh,
        out_type=jax.ShapeDtypeStruct((n_vb * band_words,), jnp.float32),
        scratch_types=[pltpu.VMEM((D, TV), jnp.float32)] * NB
        + [pltpu.VMEM((band_words,), jnp.float32)] * NB
        + [pltpu.SemaphoreType.DMA] * (2 * NB),
        compiler_params=pltpu.CompilerParams(needs_layout_passes=False),
    )
    def detile_kernel(t_hbm, r_hbm, *scratch):
        tiles = scratch[:NB]
        buft = scratch[NB : 2 * NB]
        isem = scratch[2 * NB : 3 * NB]
        wsem = scratch[3 * NB :]
        wid = lax.axis_index("s") * NC + lax.axis_index("c")

        def vb_of(j):
            return jnp.minimum(wid + j * NW, n_full - 1)

        def start_fill(j, b):
            pltpu.async_copy(
                t_hbm.at[:, pl.ds(vb_of(j) * TV, TV)], tiles[b], isem[b]
            )

        def wait_fill(b):
            pltpu.make_async_copy(
                t_hbm.at[:, pl.ds(0, TV)], tiles[b], isem[b]
            ).wait()

        def start_write(j, b):
            pltpu.async_copy(
                buft[b],
                r_hbm.at[pl.ds(vb_of(j) * band_words, band_words)],
                wsem[b],
            )

        def wait_write(b):
            pltpu.make_async_copy(
                buft[b], r_hbm.at[pl.ds(0, band_words)], wsem[b]
            ).wait()

        vvm = [lax.rem(d + _iota16(), L) for d in range(L)]
        sca = [vvm[d] * D + _iota16() for d in range(L)]

        def transpose_band(bs, n_v):
            def blk(v16, c):
                v0 = v16 * L
                for f0 in range(0, D, L):
                    rows = _iota16() + f0
                    for d in range(L):
                        vals = plsc.load_gather(
                            tiles[bs], [rows, vvm[d] + v0]
                        )
                        plsc.store_scatter(
                            buft[bs], [sca[d] + (v0 * D + f0)], vals
                        )
                return c

            lax.fori_loop(0, n_v // L, blk, 0)

        for b in range(NB):
            start_fill(b, b)

        def body(j, carry):
            bsel = lax.rem(j, NB)

            def for_buf(bs):
                @pl.when(bsel == bs)
                def _():
                    wait_fill(bs)

                    @pl.when(j >= NB)
                    def _():
                        wait_write(bs)

                    transpose_band(bs, TV)

                    @pl.when(j + NB < n_iter)
                    def _():
                        start_fill(j + NB, bs)

                    start_write(j, bs)

            for bs in range(NB):
                for_buf(bs)
            return carry

        lax.fori_loop(0, n_iter, body, 0)
        for b in range(NB):
            wait_write(b)

    return detile_kernel


@functools.lru_cache(maxsize=None)
def _make_tail_fix(n_r: int, off: int, n_tail: int):
    def tail_kernel(r_in, tail_in, r_out, tmp, s1, s2):
        cp1 = pltpu.make_async_copy(tail_in, tmp, s1)
        cp1.start()
        cp1.wait()
        cp2 = pltpu.make_async_copy(tmp, r_out.at[pl.ds(off, n_tail)], s2)
        cp2.start()
        cp2.wait()

    return pl.pallas_call(
        tail_kernel,
        out_shape=jax.ShapeDtypeStruct((n_r,), jnp.float32),
        in_specs=[
            pl.BlockSpec(memory_space=pl.ANY),
            pl.BlockSpec(memory_space=pl.ANY),
        ],
        out_specs=pl.BlockSpec(memory_space=pl.ANY),
        input_output_aliases={0: 0},
        scratch_shapes=[
            pltpu.VMEM((n_tail,), jnp.float32),
            pltpu.SemaphoreType.DMA,
            pltpu.SemaphoreType.DMA,
        ],
    )


@functools.lru_cache(maxsize=None)
def _make_gather(Vpad: int, D: int, T: int, B0: int):
    info = plsc.get_sparse_core_info()
    NC, NS = info.num_cores, info.num_subcores
    NW = NC * NS
    n_bb = B0 // TV
    n_chunks = (T * n_bb) // NW
    NB = 2
    n_fb = D // TF

    mesh = plsc.VectorSubcoreMesh(core_axis_name="c", subcore_axis_name="s")

    @functools.partial(
        pl.kernel,
        mesh=mesh,
        out_type=jax.ShapeDtypeStruct((T, n_fb, n_bb, TF, TV), jnp.float32),
        scratch_types=[
            pltpu.VMEM((NB, TV), jnp.int32),
            pltpu.VMEM((NB, TV, D), jnp.float32),
            pltpu.VMEM((NB, n_fb, TF, TV), jnp.float32),
        ]
        + [pltpu.SemaphoreType.DMA] * (3 * NB),
        compiler_params=pltpu.CompilerParams(
            use_tc_tiling_on_sc=False, needs_layout_passes=False
        ),
    )
    def gather_kernel(x4_hbm, r_hbm, out_hbm, idx_v, g, gt, *sems):
        xsem = sems[:NB]
        gsem = sems[NB : 2 * NB]
        wsem = sems[2 * NB :]
        wid = lax.axis_index("s") * NC + lax.axis_index("c")
        base = wid * n_chunks

        def tb_of(i):
            c = base + i
            return lax.div(c, n_bb), lax.rem(c, n_bb)

        def start_idx(i, b):
            t, bb = tb_of(i)
            pltpu.async_copy(
                x4_hbm.at[lax.shift_right_logical(t, 3), bb, lax.rem(t, TF)],
                idx_v.at[b],
                xsem[b],
            )

        def wait_idx(b):
            pltpu.make_async_copy(
                x4_hbm.at[0, 0, 0], idx_v.at[b], xsem[b]
            ).wait()

        def start_gather(b):
            pltpu.async_copy(r_hbm.at[idx_v.at[b]], g.at[b], gsem[b])

        def wait_gather(b):
            pltpu.make_async_copy(
                r_hbm.at[idx_v.at[b]], g.at[b], gsem[b]
            ).wait()

        def start_write(i, b):
            t, bb = tb_of(i)
            pltpu.async_copy(gt.at[b], out_hbm.at[t, :, bb], wsem[b])

        def wait_write(b):
            pltpu.make_async_copy(
                gt.at[b], out_hbm.at[0, :, 0], wsem[b]
            ).wait()

        for b in range(NB):
            start_idx(b, b)
        for b in range(NB):
            wait_idx(b)
            start_gather(b)

        vvm = [lax.rem(d + _iota16(), L) for d in range(L)]

        def transpose_chunk(bs):
            def blk(v16, c):
                v0 = v16 * L
                rows = _iota16() + v0
                for f0 in range(0, D, L):
                    for d in range(L):
                        fvec = vvm[d] + f0
                        vals = plsc.load_gather(g.at[bs], [rows, fvec])
                        plsc.store_scatter(
                            gt.at[bs],
                            [
                                lax.shift_right_logical(fvec, 3),
                                lax.rem(fvec, TF),
                                rows,
                            ],
                            vals,
                        )
                return c

            lax.fori_loop(0, TV // L, blk, 0)

        def body(i, carry):
            bsel = lax.rem(i, NB)

            def for_buf(bs):
                @pl.when(bsel == bs)
                def _():
                    wait_gather(bs)

                    @pl.when(i + NB < n_chunks)
                    def _():
                        start_idx(i + NB, bs)

                    @pl.when(i >= NB)
                    def _():
                        wait_write(bs)

                    transpose_chunk(bs)

                    @pl.when(i + NB < n_chunks)
                    def _():
                        wait_idx(bs)
                        start_gather(bs)

                    start_write(i, bs)

            for bs in range(NB):
                for_buf(bs)
            return carry

        lax.fori_loop(0, n_chunks, body, 0)
        for b in range(NB):
            wait_write(b)

    return gather_kernel


@jax.jit
def kernel(x, table):
    n_rows, n_cols = x.shape
    V, D = table.shape
    n_vb = -(-V // TV)
    t_phys = table.T
    x4 = (
        x.T.astype(jnp.int32)
        .reshape(n_cols // TF, TF, n_rows // TV, TV)
        .transpose(0, 2, 1, 3)
    )
    n_full = V // TV
    r_flat = _make_detile(V, D)(t_phys)
    if V % TV:
        tail = lax.slice(table, (n_full * TV, 0), (V, D)).reshape(-1)
        r_flat = _make_tail_fix(
            r_flat.shape[0], n_full * TV * D, tail.shape[0]
        )(r_flat, tail)
    r = r_flat.reshape(n_vb * TV, D)
    out5 = _make_gather(n_vb * TV, D, n_cols, n_rows)(x4, r)
    return out5.transpose(2, 4, 0, 1, 3).reshape(n_rows, n_cols, D)
